# Initial kernel scaffold; baseline (speedup 1.0000x reference)
import jax
import jax.numpy as jnp
from jax.experimental import pallas as pl


def kernel(inputs):
    raise NotImplementedError("write your pallas kernel here")


# speedup vs baseline: 3.8832x; 3.8832x over previous
import functools

import jax
import jax.numpy as jnp
from jax import lax
from jax.experimental import pallas as pl
from jax.experimental.pallas import tpu as pltpu
from jax.experimental.pallas import tpu_sc as plsc

CROP_H = 192
CROP_W = 192
NUM_CROPS = 4
ENV_W = CROP_W + 8


def kernel(inputs):
    B, C, H, W = inputs.shape
    max_y = H - CROP_H
    max_x = W - CROP_W
    key = jax.random.key(42)
    ky, kx = jax.random.split(key)
    rand_y = (jax.random.uniform(ky, (B, NUM_CROPS)) * max_y).astype(jnp.int32)
    rand_x = (jax.random.uniform(kx, (B, NUM_CROPS)) * max_x).astype(jnp.int32)

    T = B * NUM_CROPS * C
    t = jnp.arange(T, dtype=jnp.int32)
    bn = t // C
    c = t % C
    b = bn // NUM_CROPS
    row0 = (b * C + c) * H + rand_y.reshape(-1)[bn]
    xoff = rand_x.reshape(-1)[bn]
    xa = (xoff // 8) * 8
    xs = xoff - xa

    NW = 32
    TPW = T // NW

    def bcast(v):
        return jnp.broadcast_to(
            v.reshape(NW, TPW, 1), (NW, TPW, 16)).astype(jnp.int32)

    in_rows = inputs.reshape(B * C * H, W)

    mesh = plsc.VectorSubcoreMesh(core_axis_name="c", subcore_axis_name="s")

    @functools.partial(
        pl.kernel,
        mesh=mesh,
        compiler_params=pltpu.CompilerParams(
            use_tc_tiling_on_sc=False, needs_layout_passes=False),
        out_type=jax.ShapeDtypeStruct((T * CROP_H, CROP_W), jnp.float32),
        scratch_types=[
            pltpu.VMEM((TPW, 16), jnp.int32),
            pltpu.VMEM((TPW, 16), jnp.int32),
            pltpu.VMEM((TPW, 16), jnp.int32),
            pltpu.VMEM((CROP_H, ENV_W), jnp.float32),
            pltpu.VMEM((CROP_H, CROP_W), jnp.float32),
        ],
    )
    def crop_kernel(in_hbm, row0_hbm, xa_hbm, xs_hbm, out_hbm,
                    row0_v, xa_v, xs_v, env_v, crop_v):
        nc = 2
        wid = lax.axis_index("s") * nc + lax.axis_index("c")
        pltpu.sync_copy(row0_hbm.at[wid], row0_v)
        pltpu.sync_copy(xa_hbm.at[wid], xa_v)
        pltpu.sync_copy(xs_hbm.at[wid], xs_v)
        for k in range(TPW):
            r0 = jnp.max(row0_v[k, :])
            xak = pl.multiple_of(jnp.max(xa_v[k, :]), 8)
            xsk = jnp.max(xs_v[k, :])
            pltpu.sync_copy(
                in_hbm.at[pl.ds(r0, CROP_H), pl.ds(xak, ENV_W)], env_v)

            def row_body(i, carry):
                for j in range(CROP_W // 16):
                    crop_v[i, pl.ds(16 * j, 16)] = (
                        env_v[i, pl.ds(xsk + 16 * j, 16)])
                return carry

            lax.fori_loop(0, CROP_H, row_body, 0)
            pltpu.sync_copy(
                crop_v,
                out_hbm.at[pl.ds((wid * TPW + k) * CROP_H, CROP_H), :])

    out = crop_kernel(in_rows, bcast(row0), bcast(xa), bcast(xs))
    return out.reshape(B * NUM_CROPS, C, CROP_H, CROP_W)


# speedup vs baseline: 4.2298x; 1.0893x over previous
import functools

import jax
import jax.numpy as jnp
from jax import lax
from jax.experimental import pallas as pl
from jax.experimental.pallas import tpu as pltpu
from jax.experimental.pallas import tpu_sc as plsc

CROP_H = 192
CROP_W = 192
NUM_CROPS = 4
NW = 32


def kernel(inputs):
    B, C, H, W = inputs.shape
    max_y = H - CROP_H
    max_x = W - CROP_W
    key = jax.random.key(42)
    ky, kx = jax.random.split(key)
    rand_y = (jax.random.uniform(ky, (B, NUM_CROPS)) * max_y).astype(jnp.int32)
    rand_x = (jax.random.uniform(kx, (B, NUM_CROPS)) * max_x).astype(jnp.int32)

    G = B * C
    IPW = G // NW
    KPW = IPW * NUM_CROPS

    g = jnp.arange(G, dtype=jnp.int32)
    y_meta = rand_y[g // C]
    x_meta = rand_x[g // C]

    def bcast(v):
        return jnp.broadcast_to(
            v.reshape(NW, KPW, 1), (NW, KPW, 16)).astype(jnp.int32)

    in_rows = inputs.reshape(G * H, W)
    T = B * NUM_CROPS * C

    mesh = plsc.VectorSubcoreMesh(core_axis_name="c", subcore_axis_name="s")

    @functools.partial(
        pl.kernel,
        mesh=mesh,
        compiler_params=pltpu.CompilerParams(
            use_tc_tiling_on_sc=False, needs_layout_passes=False),
        out_type=jax.ShapeDtypeStruct((T * CROP_H, CROP_W), jnp.float32),
        scratch_types=[
            pltpu.VMEM((KPW, 16), jnp.int32),
            pltpu.VMEM((KPW, 16), jnp.int32),
            pltpu.VMEM((H, W), jnp.float32),
            pltpu.VMEM((CROP_H, CROP_W), jnp.float32),
        ],
    )
    def crop_kernel(in_hbm, y_hbm, x_hbm, out_hbm, y_v, x_v, img_v, crop_v):
        nc = 2
        wid = lax.axis_index("s") * nc + lax.axis_index("c")
        pltpu.sync_copy(y_hbm.at[wid], y_v)
        pltpu.sync_copy(x_hbm.at[wid], x_v)
        for m in range(IPW):
            pltpu.sync_copy(
                in_hbm.at[pl.ds((wid * IPW + m) * H, H), :], img_v)
            for n in range(NUM_CROPS):
                k = m * NUM_CROPS + n
                y = jnp.max(y_v[k, :])
                x = jnp.max(x_v[k, :])

                def row_body(i, carry):
                    for j in range(CROP_W // 16):
                        crop_v[i, pl.ds(16 * j, 16)] = (
                            img_v[y + i, pl.ds(x + 16 * j, 16)])
                    return carry

                lax.fori_loop(0, CROP_H, row_body, 0)
                t = wid * KPW + (m // C) * NUM_CROPS * C + n * C + m % C
                pltpu.sync_copy(
                    crop_v, out_hbm.at[pl.ds(t * CROP_H, CROP_H), :])

    out = crop_kernel(in_rows, bcast(y_meta), bcast(x_meta))
    return out.reshape(B * NUM_CROPS, C, CROP_H, CROP_W)


# speedup vs baseline: 6.2228x; 1.4712x over previous
import functools

import jax
import jax.numpy as jnp
from jax import lax
from jax.experimental import pallas as pl
from jax.experimental.pallas import tpu as pltpu
from jax.experimental.pallas import tpu_sc as plsc

CROP_H = 192
CROP_W = 192
NUM_CROPS = 4
NW = 32


def kernel(inputs):
    B, C, H, W = inputs.shape
    max_y = H - CROP_H
    max_x = W - CROP_W
    key = jax.random.key(42)
    ky, kx = jax.random.split(key)
    rand_y = (jax.random.uniform(ky, (B, NUM_CROPS)) * max_y).astype(jnp.int32)
    rand_x = (jax.random.uniform(kx, (B, NUM_CROPS)) * max_x).astype(jnp.int32)

    G = B * C
    IPW = G // NW
    KPW = IPW * NUM_CROPS

    g = jnp.arange(G, dtype=jnp.int32)
    y_meta = rand_y[g // C]
    x_meta = rand_x[g // C]

    def bcast(v):
        return jnp.broadcast_to(
            v.reshape(NW, KPW, 1), (NW, KPW, 16)).astype(jnp.int32)

    in_rows = inputs.reshape(G * H, W)
    T = B * NUM_CROPS * C

    mesh = plsc.VectorSubcoreMesh(core_axis_name="c", subcore_axis_name="s")

    @functools.partial(
        pl.kernel,
        mesh=mesh,
        compiler_params=pltpu.CompilerParams(
            use_tc_tiling_on_sc=False, needs_layout_passes=False),
        out_type=jax.ShapeDtypeStruct((T * CROP_H, CROP_W), jnp.float32),
        scratch_types=[
            pltpu.VMEM((KPW, 16), jnp.int32),
            pltpu.VMEM((KPW, 16), jnp.int32),
            pltpu.VMEM((H, W), jnp.float32),
            pltpu.VMEM((CROP_H, CROP_W), jnp.float32),
        ],
    )
    def crop_kernel(in_hbm, y_hbm, x_hbm, out_hbm, y_v, x_v, img_v, crop_v):
        nc = 2
        wid = lax.axis_index("s") * nc + lax.axis_index("c")
        pltpu.sync_copy(y_hbm.at[wid], y_v)
        pltpu.sync_copy(x_hbm.at[wid], x_v)
        for m in range(IPW):
            pltpu.sync_copy(
                in_hbm.at[pl.ds((wid * IPW + m) * H, H), :], img_v)
            for n in range(NUM_CROPS):
                k = m * NUM_CROPS + n
                y = jnp.max(y_v[k, :])
                x = jnp.max(x_v[k, :])

                @plsc.parallel_loop(0, CROP_H, unroll=4)
                def row_body(i):
                    for j in range(CROP_W // 16):
                        crop_v[i, pl.ds(16 * j, 16)] = (
                            img_v[y + i, pl.ds(x + 16 * j, 16)])
                t = wid * KPW + (m // C) * NUM_CROPS * C + n * C + m % C
                pltpu.sync_copy(
                    crop_v, out_hbm.at[pl.ds(t * CROP_H, CROP_H), :])

    out = crop_kernel(in_rows, bcast(y_meta), bcast(x_meta))
    return out.reshape(B * NUM_CROPS, C, CROP_H, CROP_W)


# speedup vs baseline: 6.5877x; 1.0586x over previous
import functools

import jax
import jax.numpy as jnp
from jax import lax
from jax.experimental import pallas as pl
from jax.experimental.pallas import tpu as pltpu
from jax.experimental.pallas import tpu_sc as plsc

CROP_H = 192
CROP_W = 192
NUM_CROPS = 4
NW = 32


def kernel(inputs):
    B, C, H, W = inputs.shape
    max_y = H - CROP_H
    max_x = W - CROP_W
    key = jax.random.key(42)
    ky, kx = jax.random.split(key)
    rand_y = (jax.random.uniform(ky, (B, NUM_CROPS)) * max_y).astype(jnp.int32)
    rand_x = (jax.random.uniform(kx, (B, NUM_CROPS)) * max_x).astype(jnp.int32)

    G = B * C
    IPW = G // NW
    KPW = IPW * NUM_CROPS

    g = jnp.arange(G, dtype=jnp.int32)
    y_meta = rand_y[g // C]
    x_meta = rand_x[g // C]

    def bcast(v):
        return jnp.broadcast_to(
            v.reshape(NW, KPW, 1), (NW, KPW, 16)).astype(jnp.int32)

    in_rows = inputs.reshape(G * H, W)
    T = B * NUM_CROPS * C

    mesh = plsc.VectorSubcoreMesh(core_axis_name="c", subcore_axis_name="s")

    @functools.partial(
        pl.kernel,
        mesh=mesh,
        compiler_params=pltpu.CompilerParams(
            use_tc_tiling_on_sc=False, needs_layout_passes=False),
        out_type=jax.ShapeDtypeStruct((T * CROP_H, CROP_W), jnp.float32),
        scratch_types=[
            pltpu.VMEM((KPW, 16), jnp.int32),
            pltpu.VMEM((KPW, 16), jnp.int32),
            pltpu.VMEM((H, W), jnp.float32),
            pltpu.VMEM((CROP_H, CROP_W), jnp.float32),
            pltpu.VMEM((CROP_H, CROP_W), jnp.float32),
            pltpu.SemaphoreType.DMA,
            pltpu.SemaphoreType.DMA,
        ],
    )
    def crop_kernel(in_hbm, y_hbm, x_hbm, out_hbm, y_v, x_v, img_v,
                    crop0_v, crop1_v, sem0, sem1):
        nc = 2
        wid = lax.axis_index("s") * nc + lax.axis_index("c")
        pltpu.sync_copy(y_hbm.at[wid], y_v)
        pltpu.sync_copy(x_hbm.at[wid], x_v)
        crops = (crop0_v, crop1_v)
        sems = (sem0, sem1)
        pending = [None, None]
        for m in range(IPW):
            pltpu.sync_copy(
                in_hbm.at[pl.ds((wid * IPW + m) * H, H), :], img_v)
            for n in range(NUM_CROPS):
                k = m * NUM_CROPS + n
                buf = k % 2
                y = jnp.max(y_v[k, :])
                x = jnp.max(x_v[k, :])
                if pending[buf] is not None:
                    pending[buf].wait()
                crop_v = crops[buf]

                @plsc.parallel_loop(0, CROP_H, unroll=4)
                def row_body(i):
                    for j in range(CROP_W // 16):
                        crop_v[i, pl.ds(16 * j, 16)] = (
                            img_v[y + i, pl.ds(x + 16 * j, 16)])
                t = wid * KPW + (m // C) * NUM_CROPS * C + n * C + m % C
                pending[buf] = pltpu.async_copy(
                    crop_v, out_hbm.at[pl.ds(t * CROP_H, CROP_H), :],
                    sems[buf])
        pending[0].wait()
        pending[1].wait()

    out = crop_kernel(in_rows, bcast(y_meta), bcast(x_meta))
    return out.reshape(B * NUM_CROPS, C, CROP_H, CROP_W)


# speedup vs baseline: 6.5914x; 1.0006x over previous
import functools

import jax
import jax.numpy as jnp
from jax import lax
from jax.experimental import pallas as pl
from jax.experimental.pallas import tpu as pltpu
from jax.experimental.pallas import tpu_sc as plsc

CROP_H = 192
CROP_W = 192
NUM_CROPS = 4
NW = 32


def kernel(inputs):
    B, C, H, W = inputs.shape
    max_y = H - CROP_H
    max_x = W - CROP_W
    key = jax.random.key(42)
    ky, kx = jax.random.split(key)
    rand_y = (jax.random.uniform(ky, (B, NUM_CROPS)) * max_y).astype(jnp.int32)
    rand_x = (jax.random.uniform(kx, (B, NUM_CROPS)) * max_x).astype(jnp.int32)

    G = B * C
    IPW = G // NW
    KPW = IPW * NUM_CROPS

    g = jnp.arange(G, dtype=jnp.int32)
    y_meta = rand_y[g // C]
    x_meta = rand_x[g // C]

    def bcast(v):
        return jnp.broadcast_to(
            v.reshape(NW, KPW, 1), (NW, KPW, 16)).astype(jnp.int32)

    in_flat = inputs.reshape(G * H * W)
    T = B * NUM_CROPS * C
    CROP_SZ = CROP_H * CROP_W

    mesh = plsc.VectorSubcoreMesh(core_axis_name="c", subcore_axis_name="s")

    @functools.partial(
        pl.kernel,
        mesh=mesh,
        compiler_params=pltpu.CompilerParams(
            use_tc_tiling_on_sc=False, needs_layout_passes=False),
        out_type=jax.ShapeDtypeStruct((T * CROP_SZ,), jnp.float32),
        scratch_types=[
            pltpu.VMEM((KPW, 16), jnp.int32),
            pltpu.VMEM((KPW, 16), jnp.int32),
            pltpu.VMEM((H * W,), jnp.float32),
            pltpu.VMEM((CROP_SZ,), jnp.float32),
            pltpu.VMEM((CROP_SZ,), jnp.float32),
            pltpu.SemaphoreType.DMA,
            pltpu.SemaphoreType.DMA,
        ],
    )
    def crop_kernel(in_hbm, y_hbm, x_hbm, out_hbm, y_v, x_v, img_v,
                    crop0_v, crop1_v, sem0, sem1):
        nc = 2
        wid = lax.axis_index("s") * nc + lax.axis_index("c")
        pltpu.sync_copy(y_hbm.at[wid], y_v)
        pltpu.sync_copy(x_hbm.at[wid], x_v)
        crops = (crop0_v, crop1_v)
        sems = (sem0, sem1)
        pending = [None, None]
        for m in range(IPW):
            pltpu.sync_copy(
                in_hbm.at[pl.ds((wid * IPW + m) * (H * W), H * W)], img_v)
            for n in range(NUM_CROPS):
                k = m * NUM_CROPS + n
                buf = k % 2
                y = jnp.max(y_v[k, :])
                x = jnp.max(x_v[k, :])
                if pending[buf] is not None:
                    pending[buf].wait()
                crop_v = crops[buf]
                base = y * W + x

                @plsc.parallel_loop(0, CROP_H, unroll=4)
                def row_body(i):
                    for j in range(CROP_W // 16):
                        crop_v[pl.ds(i * CROP_W + 16 * j, 16)] = (
                            img_v[pl.ds(base + i * W + 16 * j, 16)])
                t = wid * KPW + (m // C) * NUM_CROPS * C + n * C + m % C
                pending[buf] = pltpu.async_copy(
                    crop_v, out_hbm.at[pl.ds(t * CROP_SZ, CROP_SZ)],
                    sems[buf])
        pending[0].wait()
        pending[1].wait()

    out = crop_kernel(in_flat, bcast(y_meta), bcast(x_meta))
    return out.reshape(B * NUM_CROPS, C, CROP_H, CROP_W)
